# baseline (speedup 1.0000x reference)
import functools

import jax
import jax.numpy as jnp
from jax import lax
from jax.experimental import pallas as pl
from jax.experimental.pallas import tpu as pltpu
from jax.experimental.pallas import tpu_sc as plsc

N_NODES = 50000
C = 32
NPAD = 50176
E = 800000
CHUNK = 128
NCHUNKS = 6272
EPAD = NCHUNKS * CHUNK
NCORES, NSUB = 2, 16
CH_PER_SC = NCHUNKS // NCORES
CH_PER_TILE = CH_PER_SC // NSUB
ROWS_PER_TILE = NPAD // NSUB
GRP = 14
NGRP = CH_PER_TILE // GRP
NBUF = 4
ZB = 196

_mesh = plsc.VectorSubcoreMesh(
    core_axis_name="c", subcore_axis_name="s",
    num_cores=NCORES, num_subcores=NSUB)
_sc_params = pltpu.CompilerParams(use_tc_tiling_on_sc=False,
                                  needs_layout_passes=False)


@functools.partial(
    pl.kernel,
    out_type=jax.ShapeDtypeStruct((NCORES * NSUB, NPAD), jnp.float32),
    mesh=_mesh,
    compiler_params=_sc_params,
    scratch_types=[
        pltpu.VMEM((CH_PER_TILE, CHUNK), jnp.int32),
        pltpu.VMEM((NPAD,), jnp.float32),
    ],
)
def _sc_degree(edges, out, didx, deg_v):
    cid = lax.axis_index("c")
    sid = lax.axis_index("s")
    wid = cid * NSUB + sid
    ch0 = cid * CH_PER_SC + sid * CH_PER_TILE
    pltpu.sync_copy(edges.at[1, pl.ds(ch0, CH_PER_TILE), :], didx)
    zero16 = jnp.zeros((16,), jnp.float32)

    def zbody(i, carry):
        deg_v[pl.ds(i * 16, 16)] = zero16
        return carry

    lax.fori_loop(0, NPAD // 16, zbody, 0)
    one16 = jnp.ones((16,), jnp.float32)

    def body(j, carry):
        for k in range(CHUNK // 16):
            idx = didx[j, pl.ds(k * 16, 16)]
            plsc.addupdate_scatter(deg_v, [idx], one16)
        return carry

    lax.fori_loop(0, CH_PER_TILE, body, 0)
    pltpu.sync_copy(deg_v, out.at[wid, :])


@functools.partial(
    pl.kernel,
    out_type=jax.ShapeDtypeStruct((NCORES, NPAD, C), jnp.float32),
    mesh=_mesh,
    compiler_params=_sc_params,
    scratch_types=[
        pltpu.VMEM((2, GRP, CHUNK), jnp.int32),
        pltpu.VMEM((2, GRP, CHUNK), jnp.int32),
        [pltpu.VMEM((CHUNK, C), jnp.float32)] * NBUF,
        pltpu.VMEM((ZB, C), jnp.float32),
        pltpu.VMEM_SHARED((NPAD, C), jnp.float32),
        pltpu.SemaphoreType.DMA,
        [pltpu.SemaphoreType.DMA] * NBUF,
        [pltpu.SemaphoreType.DMA] * NBUF,
    ],
)
def _sc_prop(edges, table, zer_h, out, sidxb, didxb, rows, zb,
             acc_sh, sem_i, gsems, ssems):
    cid = lax.axis_index("c")
    sid = lax.axis_index("s")
    base = sid * ROWS_PER_TILE
    ch0 = cid * CH_PER_SC + sid * CH_PER_TILE

    def idx_start(g, pb):
        pltpu.async_copy(edges.at[0, pl.ds(ch0 + g * GRP, GRP), :],
                         sidxb.at[pb], sem_i)
        pltpu.async_copy(edges.at[1, pl.ds(ch0 + g * GRP, GRP), :],
                         didxb.at[pb], sem_i)

    def idx_wait(g, pb):
        pltpu.make_async_copy(edges.at[0, pl.ds(ch0 + g * GRP, GRP), :],
                              sidxb.at[pb], sem_i).wait()
        pltpu.make_async_copy(edges.at[1, pl.ds(ch0 + g * GRP, GRP), :],
                              didxb.at[pb], sem_i).wait()

    idx_start(0, 0)
    for k in range(ROWS_PER_TILE // ZB):
        pltpu.sync_copy(zer_h, acc_sh.at[pl.ds(base + k * ZB, ZB), :])
    plsc.subcore_barrier()

    def group_body(g, carry):
        pb = lax.rem(g, 2)
        idx_wait(g, pb)

        @pl.when(g + 1 < NGRP)
        def _():
            idx_start(g + 1, 1 - pb)

        gd = [None] * NBUF
        sd = [None] * NBUF
        for jj in range(GRP):
            b = jj % NBUF
            if sd[b] is not None:
                sd[b].wait()
                sd[b] = None
            gd[b] = pltpu.async_copy(
                table.at[sidxb.at[pb, jj]], rows[b], gsems[b])
            if jj > 0:
                b1 = (jj - 1) % NBUF
                gd[b1].wait()
                sd[b1] = pltpu.async_copy(
                    rows[b1], acc_sh.at[didxb.at[pb, jj - 1]], ssems[b1],
                    add=True)
        bl = (GRP - 1) % NBUF
        gd[bl].wait()
        sd[bl] = pltpu.async_copy(
            rows[bl], acc_sh.at[didxb.at[pb, GRP - 1]], ssems[bl], add=True)
        for b in range(NBUF):
            if sd[b] is not None:
                sd[b].wait()
        return carry

    lax.fori_loop(0, NGRP, group_body, 0)
    plsc.subcore_barrier()
    pltpu.sync_copy(acc_sh.at[pl.ds(base, ROWS_PER_TILE), :],
                    out.at[cid, pl.ds(base, ROWS_PER_TILE), :])


TB = 3584


def _tca1_body(x_ref, pe_ref, moh_ref, m2_ref, g_ref):
    x = x_ref[...]
    io = lax.broadcasted_iota(jnp.int32, (TB, C), 1)
    oh = (x == io).astype(jnp.float32)
    g = jnp.dot(oh, moh_ref[...], preferred_element_type=jnp.float32)
    g_ref[...] = g + jnp.dot(pe_ref[...], m2_ref[...],
                             preferred_element_type=jnp.float32)


def _tc_a1(x_pad, pe8, moh, m2):
    return pl.pallas_call(
        _tca1_body,
        grid=(NPAD // TB,),
        in_specs=[
            pl.BlockSpec((TB, 1), lambda i: (i, 0)),
            pl.BlockSpec((TB, 8), lambda i: (i, 0)),
            pl.BlockSpec((C, C), lambda i: (0, 0)),
            pl.BlockSpec((8, C), lambda i: (0, 0)),
        ],
        out_specs=pl.BlockSpec((TB, C), lambda i: (i, 0)),
        out_shape=jax.ShapeDtypeStruct((NPAD, C), jnp.float32),
    )(x_pad, pe8, moh, m2)


def _tca2_body(g_ref, dg_ref, gs_ref, dinv_ref):
    deg = lax.dot_general(
        dg_ref[...], jnp.ones((NCORES * NSUB, 1), jnp.float32),
        (((0,), (0,)), ((), ())), preferred_element_type=jnp.float32) + 1.0
    dinv = lax.rsqrt(deg)
    dinv_ref[...] = dinv
    gs_ref[...] = g_ref[...] * dinv


def _tc_a2(g1, degp):
    return pl.pallas_call(
        _tca2_body,
        grid=(NPAD // TB,),
        in_specs=[
            pl.BlockSpec((TB, C), lambda i: (i, 0)),
            pl.BlockSpec((NCORES * NSUB, TB), lambda i: (0, i)),
        ],
        out_specs=[
            pl.BlockSpec((TB, C), lambda i: (i, 0)),
            pl.BlockSpec((TB, 1), lambda i: (i, 0)),
        ],
        out_shape=[
            jax.ShapeDtypeStruct((NPAD, C), jnp.float32),
            jax.ShapeDtypeStruct((NPAD, 1), jnp.float32),
        ],
    )(g1, degp)


def _tcb_body(a_ref, gs_ref, dinv_ref, b1_ref, hs_ref):
    s = a_ref[0] + a_ref[1] + gs_ref[...]
    z = s * dinv_ref[...] + b1_ref[0:1, :]
    hs_ref[...] = jnp.maximum(z, 0.0) * dinv_ref[...]


def _tc_b(acc, gs, dinv, b1m):
    return pl.pallas_call(
        _tcb_body,
        grid=(NPAD // TB,),
        in_specs=[
            pl.BlockSpec((NCORES, TB, C), lambda i: (0, i, 0)),
            pl.BlockSpec((TB, C), lambda i: (i, 0)),
            pl.BlockSpec((TB, 1), lambda i: (i, 0)),
            pl.BlockSpec((8, C), lambda i: (0, 0)),
        ],
        out_specs=pl.BlockSpec((TB, C), lambda i: (i, 0)),
        out_shape=jax.ShapeDtypeStruct((NPAD, C), jnp.float32),
    )(acc, gs, dinv, b1m)


def _tcc_body(a_ref, hs_ref, dinv_ref, wml_ref, bml_ref, o_ref):
    p = (a_ref[0] + a_ref[1] + hs_ref[...]) * dinv_ref[...]
    o_ref[...] = (jnp.dot(p, wml_ref[...], preferred_element_type=jnp.float32)
                  + bml_ref[0:1, :])


def _tc_c(acc, hs, dinv, wml, bml):
    return pl.pallas_call(
        _tcc_body,
        grid=(NPAD // TB,),
        in_specs=[
            pl.BlockSpec((NCORES, TB, C), lambda i: (0, i, 0)),
            pl.BlockSpec((TB, C), lambda i: (i, 0)),
            pl.BlockSpec((TB, 1), lambda i: (i, 0)),
            pl.BlockSpec((C, 2 * C), lambda i: (0, 0)),
            pl.BlockSpec((8, 2 * C), lambda i: (0, 0)),
        ],
        out_specs=pl.BlockSpec((TB, 2 * C), lambda i: (i, 0)),
        out_shape=jax.ShapeDtypeStruct((NPAD, 2 * C), jnp.float32),
    )(acc, hs, dinv, wml, bml)


def kernel(x, edge_index, laplacian_eigenvector_pe, embed_table, trans_W,
           trans_b, W1, b1, W_mu, b_mu, W_ls, b_ls):
    f32 = jnp.float32
    ei = edge_index.astype(jnp.int32)
    pad = jnp.full((2, EPAD - E), N_NODES, jnp.int32)
    edges = jnp.concatenate([ei, pad], axis=1).reshape(2, NCHUNKS, CHUNK)

    x_pad = jnp.pad(x.astype(jnp.int32).reshape(N_NODES, 1),
                    ((0, NPAD - N_NODES), (0, 0)))
    peb = jnp.pad(laplacian_eigenvector_pe.astype(f32),
                  ((0, NPAD - N_NODES), (0, 0)))
    pe8 = jnp.concatenate(
        [peb, jnp.ones((NPAD, 1), f32), jnp.zeros((NPAD, 2), f32)], axis=1)

    moh = jnp.pad(embed_table.astype(f32) @ W1, ((0, C - 28), (0, 0)))
    m2 = jnp.concatenate(
        [trans_W @ W1, (trans_b @ W1)[None, :], jnp.zeros((2, C), f32)],
        axis=0)
    b1m = jnp.pad(b1[None, :], ((0, 7), (0, 0)))
    wml = jnp.concatenate([W_mu, W_ls], axis=1)
    bml = jnp.pad(jnp.concatenate([b_mu, b_ls])[None, :], ((0, 7), (0, 0)))

    zprop = jnp.zeros((ZB, C), f32)

    degp = _sc_degree(edges)
    g1 = _tc_a1(x_pad, pe8, moh, m2)
    gs, dinv = _tc_a2(g1, degp)
    acc1 = _sc_prop(edges, gs, zprop)
    hs = _tc_b(acc1, gs, dinv, b1m)
    acc2 = _sc_prop(edges, hs, zprop)
    out = _tc_c(acc2, hs, dinv, wml, bml)
    return out[:N_NODES, :C], out[:N_NODES, C:]


# speedup vs baseline: 1.0786x; 1.0786x over previous
import functools

import jax
import jax.numpy as jnp
from jax import lax
from jax.experimental import pallas as pl
from jax.experimental.pallas import tpu as pltpu
from jax.experimental.pallas import tpu_sc as plsc

N_NODES = 50000
C = 32
NPAD = 50176
E = 800000
CHUNK = 128
NCHUNKS = 6272
EPAD = NCHUNKS * CHUNK
NCORES, NSUB = 2, 16
CH_PER_SC = NCHUNKS // NCORES
CH_PER_TILE = CH_PER_SC // NSUB
ROWS_PER_TILE = NPAD // NSUB
GRP = 14
NGRP = CH_PER_TILE // GRP
NBUF = 4
ZB = 196

_mesh = plsc.VectorSubcoreMesh(
    core_axis_name="c", subcore_axis_name="s",
    num_cores=NCORES, num_subcores=NSUB)
_sc_params = pltpu.CompilerParams(use_tc_tiling_on_sc=False,
                                  needs_layout_passes=False)


@functools.partial(
    pl.kernel,
    out_type=jax.ShapeDtypeStruct((NCORES * NSUB, NPAD), jnp.float32),
    mesh=_mesh,
    compiler_params=_sc_params,
    scratch_types=[
        pltpu.VMEM((CH_PER_TILE, CHUNK), jnp.int32),
        pltpu.VMEM((NPAD,), jnp.float32),
    ],
)
def _sc_degree(edges, out, didx, deg_v):
    cid = lax.axis_index("c")
    sid = lax.axis_index("s")
    wid = cid * NSUB + sid
    ch0 = cid * CH_PER_SC + sid * CH_PER_TILE
    pltpu.sync_copy(edges.at[1, pl.ds(ch0, CH_PER_TILE), :], didx)
    zero16 = jnp.zeros((16,), jnp.float32)

    def zbody(i, carry):
        deg_v[pl.ds(i * 16, 16)] = zero16
        return carry

    lax.fori_loop(0, NPAD // 16, zbody, 0)
    one16 = jnp.ones((16,), jnp.float32)

    def body(j, carry):
        for k in range(CHUNK // 16):
            idx = didx[j, pl.ds(k * 16, 16)]
            plsc.addupdate_scatter(deg_v, [idx], one16)
        return carry

    lax.fori_loop(0, CH_PER_TILE, body, 0)
    pltpu.sync_copy(deg_v, out.at[wid, :])


@functools.partial(
    pl.kernel,
    out_type=jax.ShapeDtypeStruct((NCORES, NPAD, C), jnp.float32),
    mesh=_mesh,
    compiler_params=_sc_params,
    scratch_types=[
        pltpu.VMEM((2, GRP, CHUNK), jnp.int32),
        pltpu.VMEM((2, GRP, CHUNK), jnp.int32),
        [pltpu.VMEM((CHUNK, C), jnp.float32)] * NBUF,
        pltpu.VMEM((ZB, C), jnp.float32),
        pltpu.VMEM_SHARED((NPAD, C), jnp.float32),
        pltpu.SemaphoreType.DMA,
        [pltpu.SemaphoreType.DMA] * NBUF,
        [pltpu.SemaphoreType.DMA] * NBUF,
    ],
)
def _sc_prop(edges, table, zer_h, out, sidxb, didxb, rows, zb,
             acc_sh, sem_i, gsems, ssems):
    cid = lax.axis_index("c")
    sid = lax.axis_index("s")
    base = sid * ROWS_PER_TILE
    ch0 = cid * CH_PER_SC + sid * CH_PER_TILE

    def idx_start(g, pb):
        pltpu.async_copy(edges.at[0, pl.ds(ch0 + g * GRP, GRP), :],
                         sidxb.at[pb], sem_i)
        pltpu.async_copy(edges.at[1, pl.ds(ch0 + g * GRP, GRP), :],
                         didxb.at[pb], sem_i)

    def idx_wait(g, pb):
        pltpu.make_async_copy(edges.at[0, pl.ds(ch0 + g * GRP, GRP), :],
                              sidxb.at[pb], sem_i).wait()
        pltpu.make_async_copy(edges.at[1, pl.ds(ch0 + g * GRP, GRP), :],
                              didxb.at[pb], sem_i).wait()

    idx_start(0, 0)
    pltpu.sync_copy(zer_h, zb)
    for k in range(ROWS_PER_TILE // ZB):
        pltpu.sync_copy(zb, acc_sh.at[pl.ds(base + k * ZB, ZB), :])
    plsc.subcore_barrier()

    def group_body(g, carry):
        pb = lax.rem(g, 2)
        idx_wait(g, pb)

        @pl.when(g + 1 < NGRP)
        def _():
            idx_start(g + 1, 1 - pb)

        gd = [None] * NBUF
        sd = [None] * NBUF
        for jj in range(GRP):
            b = jj % NBUF
            if sd[b] is not None:
                sd[b].wait()
                sd[b] = None
            gd[b] = pltpu.async_copy(
                table.at[sidxb.at[pb, jj]], rows[b], gsems[b])
            if jj > 0:
                b1 = (jj - 1) % NBUF
                gd[b1].wait()
                sd[b1] = pltpu.async_copy(
                    rows[b1], acc_sh.at[didxb.at[pb, jj - 1]], ssems[b1],
                    add=True)
        bl = (GRP - 1) % NBUF
        gd[bl].wait()
        sd[bl] = pltpu.async_copy(
            rows[bl], acc_sh.at[didxb.at[pb, GRP - 1]], ssems[bl], add=True)
        for b in range(NBUF):
            if sd[b] is not None:
                sd[b].wait()
        return carry

    lax.fori_loop(0, NGRP, group_body, 0)
    plsc.subcore_barrier()
    for k in range(ROWS_PER_TILE // ZB):
        pltpu.sync_copy(acc_sh.at[pl.ds(base + k * ZB, ZB), :], zb)
        pltpu.sync_copy(zb, out.at[cid, pl.ds(base + k * ZB, ZB), :])


TB = 3584


def _tca1_body(x_ref, pe_ref, moh_ref, m2_ref, g_ref):
    x = x_ref[...]
    io = lax.broadcasted_iota(jnp.int32, (TB, C), 1)
    oh = (x == io).astype(jnp.float32)
    g = jnp.dot(oh, moh_ref[...], preferred_element_type=jnp.float32)
    g_ref[...] = g + jnp.dot(pe_ref[...], m2_ref[...],
                             preferred_element_type=jnp.float32)


def _tc_a1(x_pad, pe8, moh, m2):
    return pl.pallas_call(
        _tca1_body,
        grid=(NPAD // TB,),
        in_specs=[
            pl.BlockSpec((TB, 1), lambda i: (i, 0)),
            pl.BlockSpec((TB, 8), lambda i: (i, 0)),
            pl.BlockSpec((C, C), lambda i: (0, 0)),
            pl.BlockSpec((8, C), lambda i: (0, 0)),
        ],
        out_specs=pl.BlockSpec((TB, C), lambda i: (i, 0)),
        out_shape=jax.ShapeDtypeStruct((NPAD, C), jnp.float32),
    )(x_pad, pe8, moh, m2)


def _tca2_body(g_ref, dg_ref, gs_ref, dinv_ref):
    deg = lax.dot_general(
        dg_ref[...], jnp.ones((NCORES * NSUB, 1), jnp.float32),
        (((0,), (0,)), ((), ())), preferred_element_type=jnp.float32) + 1.0
    dinv = lax.rsqrt(deg)
    dinv_ref[...] = dinv
    gs_ref[...] = g_ref[...] * dinv


def _tc_a2(g1, degp):
    return pl.pallas_call(
        _tca2_body,
        grid=(NPAD // TB,),
        in_specs=[
            pl.BlockSpec((TB, C), lambda i: (i, 0)),
            pl.BlockSpec((NCORES * NSUB, TB), lambda i: (0, i)),
        ],
        out_specs=[
            pl.BlockSpec((TB, C), lambda i: (i, 0)),
            pl.BlockSpec((TB, 1), lambda i: (i, 0)),
        ],
        out_shape=[
            jax.ShapeDtypeStruct((NPAD, C), jnp.float32),
            jax.ShapeDtypeStruct((NPAD, 1), jnp.float32),
        ],
    )(g1, degp)


def _tcb_body(a_ref, gs_ref, dinv_ref, b1_ref, hs_ref):
    s = a_ref[0] + a_ref[1] + gs_ref[...]
    z = s * dinv_ref[...] + b1_ref[0:1, :]
    hs_ref[...] = jnp.maximum(z, 0.0) * dinv_ref[...]


def _tc_b(acc, gs, dinv, b1m):
    return pl.pallas_call(
        _tcb_body,
        grid=(NPAD // TB,),
        in_specs=[
            pl.BlockSpec((NCORES, TB, C), lambda i: (0, i, 0)),
            pl.BlockSpec((TB, C), lambda i: (i, 0)),
            pl.BlockSpec((TB, 1), lambda i: (i, 0)),
            pl.BlockSpec((8, C), lambda i: (0, 0)),
        ],
        out_specs=pl.BlockSpec((TB, C), lambda i: (i, 0)),
        out_shape=jax.ShapeDtypeStruct((NPAD, C), jnp.float32),
    )(acc, gs, dinv, b1m)


def _tcc_body(a_ref, hs_ref, dinv_ref, wml_ref, bml_ref, o_ref):
    p = (a_ref[0] + a_ref[1] + hs_ref[...]) * dinv_ref[...]
    o_ref[...] = (jnp.dot(p, wml_ref[...], preferred_element_type=jnp.float32)
                  + bml_ref[0:1, :])


def _tc_c(acc, hs, dinv, wml, bml):
    return pl.pallas_call(
        _tcc_body,
        grid=(NPAD // TB,),
        in_specs=[
            pl.BlockSpec((NCORES, TB, C), lambda i: (0, i, 0)),
            pl.BlockSpec((TB, C), lambda i: (i, 0)),
            pl.BlockSpec((TB, 1), lambda i: (i, 0)),
            pl.BlockSpec((C, 2 * C), lambda i: (0, 0)),
            pl.BlockSpec((8, 2 * C), lambda i: (0, 0)),
        ],
        out_specs=pl.BlockSpec((TB, 2 * C), lambda i: (i, 0)),
        out_shape=jax.ShapeDtypeStruct((NPAD, 2 * C), jnp.float32),
    )(acc, hs, dinv, wml, bml)


def kernel(x, edge_index, laplacian_eigenvector_pe, embed_table, trans_W,
           trans_b, W1, b1, W_mu, b_mu, W_ls, b_ls):
    f32 = jnp.float32
    ei = edge_index.astype(jnp.int32)
    pad = jnp.full((2, EPAD - E), N_NODES, jnp.int32)
    edges = jnp.concatenate([ei, pad], axis=1).reshape(2, NCHUNKS, CHUNK)

    x_pad = jnp.pad(x.astype(jnp.int32).reshape(N_NODES, 1),
                    ((0, NPAD - N_NODES), (0, 0)))
    peb = jnp.pad(laplacian_eigenvector_pe.astype(f32),
                  ((0, NPAD - N_NODES), (0, 0)))
    pe8 = jnp.concatenate(
        [peb, jnp.ones((NPAD, 1), f32), jnp.zeros((NPAD, 2), f32)], axis=1)

    moh = jnp.pad(embed_table.astype(f32) @ W1, ((0, C - 28), (0, 0)))
    m2 = jnp.concatenate(
        [trans_W @ W1, (trans_b @ W1)[None, :], jnp.zeros((2, C), f32)],
        axis=0)
    b1m = jnp.pad(b1[None, :], ((0, 7), (0, 0)))
    wml = jnp.concatenate([W_mu, W_ls], axis=1)
    bml = jnp.pad(jnp.concatenate([b_mu, b_ls])[None, :], ((0, 7), (0, 0)))

    zprop = jnp.zeros((ZB, C), f32)

    degp = _sc_degree(edges)
    g1 = _tc_a1(x_pad, pe8, moh, m2)
    gs, dinv = _tc_a2(g1, degp)
    acc1 = _sc_prop(edges, gs, zprop)
    hs = _tc_b(acc1, gs, dinv, b1m)
    acc2 = _sc_prop(edges, hs, zprop)
    out = _tc_c(acc2, hs, dinv, wml, bml)
    return out[:N_NODES, :C], out[:N_NODES, C:]
